# baseline (speedup 1.0000x reference)
import functools

import jax
import jax.numpy as jnp
import numpy as np
from jax.experimental import pallas as pl
from jax.experimental.pallas import tpu as pltpu

_CHUNK = 128


def _cumsum_kernel(block_w, x_ref, t_ref, m_ref, o_ref, carry_ref, g_ref,
                   s_ref):
    k = pl.program_id(0)
    nch = block_w // _CHUNK
    c_row = nch

    @pl.when(k == 0)
    def _():
        carry_ref[...] = jnp.zeros_like(carry_ref)
        gi = jax.lax.broadcasted_iota(jnp.int32, (block_w, _CHUNK), 0)
        gc = jax.lax.broadcasted_iota(jnp.int32, (block_w, _CHUNK), 1)
        g_ref[...] = ((gi // _CHUNK) == gc).astype(jnp.float32)
        sd = jax.lax.broadcasted_iota(jnp.int32, (_CHUNK, block_w + _CHUNK), 0)
        sj = jax.lax.broadcasted_iota(jnp.int32, (_CHUNK, block_w + _CHUNK), 1)
        s_ref[...] = ((sd < jnp.minimum(sj // _CHUNK, nch))
                      | (sd == c_row)).astype(jnp.float32)

    ct = jax.lax.dot(x_ref[...], g_ref[...],
                     preferred_element_type=jnp.float32)
    ctf = ct + carry_ref[...] * m_ref[...]
    t = t_ref[...]
    for c in range(nch):
        sl = slice(c * _CHUNK, (c + 1) * _CHUNK)
        local = jax.lax.dot(x_ref[:, sl], t,
                            preferred_element_type=jnp.float32)
        carr = jax.lax.dot(ctf, s_ref[:, sl],
                           preferred_element_type=jnp.float32)
        o_ref[:, sl] = local + carr
    carry_ref[...] = jax.lax.dot(ctf, s_ref[:, block_w:block_w + _CHUNK],
                                 preferred_element_type=jnp.float32)


@jax.jit
def kernel(x):
    rows, n = x.shape
    block_w = 4096
    nch = block_w // _CHUNK
    tri = jnp.asarray(np.triu(np.ones((_CHUNK, _CHUNK), np.float32)))
    m = np.zeros((_CHUNK, _CHUNK), np.float32)
    m[:, nch] = 1.0
    m = jnp.asarray(m)
    return pl.pallas_call(
        functools.partial(_cumsum_kernel, block_w),
        grid=(n // block_w,),
        in_specs=[
            pl.BlockSpec((rows, block_w), lambda k: (0, k)),
            pl.BlockSpec((_CHUNK, _CHUNK), lambda k: (0, 0)),
            pl.BlockSpec((_CHUNK, _CHUNK), lambda k: (0, 0)),
        ],
        out_specs=pl.BlockSpec((rows, block_w), lambda k: (0, k)),
        out_shape=jax.ShapeDtypeStruct((rows, n), jnp.float32),
        scratch_shapes=[
            pltpu.VMEM((rows, _CHUNK), jnp.float32),
            pltpu.VMEM((block_w, _CHUNK), jnp.float32),
            pltpu.VMEM((_CHUNK, block_w + _CHUNK), jnp.float32),
        ],
    )(x, tri, m)


# speedup vs baseline: 1.1009x; 1.1009x over previous
import functools

import jax
import jax.numpy as jnp
import numpy as np
from jax.experimental import pallas as pl
from jax.experimental.pallas import tpu as pltpu

_CHUNK = 128


def _cumsum_kernel(block_w, x_ref, t_ref, m_ref, o_ref, carry_ref, g_ref,
                   s_ref):
    k = pl.program_id(0)
    nch = block_w // _CHUNK
    c_row = nch

    @pl.when(k == 0)
    def _():
        carry_ref[...] = jnp.zeros_like(carry_ref)
        gi = jax.lax.broadcasted_iota(jnp.int32, (block_w, _CHUNK), 0)
        gc = jax.lax.broadcasted_iota(jnp.int32, (block_w, _CHUNK), 1)
        g_ref[...] = ((gi // _CHUNK) == gc).astype(jnp.bfloat16)
        sd = jax.lax.broadcasted_iota(jnp.int32, (_CHUNK, block_w + _CHUNK), 0)
        sj = jax.lax.broadcasted_iota(jnp.int32, (_CHUNK, block_w + _CHUNK), 1)
        s_ref[...] = ((sd < jnp.minimum(sj // _CHUNK, nch))
                      | (sd == c_row)).astype(jnp.bfloat16)

    ct = jax.lax.dot(x_ref[...].astype(jnp.bfloat16), g_ref[...],
                     preferred_element_type=jnp.float32)
    ctf = (ct + carry_ref[...] * m_ref[...]).astype(jnp.bfloat16)
    t = t_ref[...]
    for c in range(nch):
        sl = slice(c * _CHUNK, (c + 1) * _CHUNK)
        local = jax.lax.dot(x_ref[:, sl].astype(jnp.bfloat16), t,
                            preferred_element_type=jnp.float32)
        carr = jax.lax.dot(ctf, s_ref[:, sl],
                           preferred_element_type=jnp.float32)
        o_ref[:, sl] = local + carr
    carry_ref[...] = jax.lax.dot(ctf, s_ref[:, block_w:block_w + _CHUNK],
                                 preferred_element_type=jnp.float32)


@jax.jit
def kernel(x):
    rows, n = x.shape
    block_w = 8192
    nch = block_w // _CHUNK
    tri = jnp.asarray(np.triu(np.ones((_CHUNK, _CHUNK), np.float32)),
                      dtype=jnp.bfloat16)
    m = np.zeros((_CHUNK, _CHUNK), np.float32)
    m[:, nch] = 1.0
    m = jnp.asarray(m)
    return pl.pallas_call(
        functools.partial(_cumsum_kernel, block_w),
        grid=(n // block_w,),
        in_specs=[
            pl.BlockSpec((rows, block_w), lambda k: (0, k)),
            pl.BlockSpec((_CHUNK, _CHUNK), lambda k: (0, 0)),
            pl.BlockSpec((_CHUNK, _CHUNK), lambda k: (0, 0)),
        ],
        out_specs=pl.BlockSpec((rows, block_w), lambda k: (0, k)),
        out_shape=jax.ShapeDtypeStruct((rows, n), jnp.float32),
        scratch_shapes=[
            pltpu.VMEM((rows, _CHUNK), jnp.float32),
            pltpu.VMEM((block_w, _CHUNK), jnp.bfloat16),
            pltpu.VMEM((_CHUNK, block_w + _CHUNK), jnp.bfloat16),
        ],
    )(x, tri, m)
